# Initial kernel scaffold; baseline (speedup 1.0000x reference)
import jax
import jax.numpy as jnp
from jax.experimental import pallas as pl


def kernel(tokens, MoE_type_tensor, wte_weight, learned_embedding_text, learned_embedding_table):
    raise NotImplementedError("write your pallas kernel here")


# speedup vs baseline: 2.6498x; 2.6498x over previous
import functools

import jax
import jax.numpy as jnp
from jax import lax
from jax.experimental import pallas as pl
from jax.experimental.pallas import tpu as pltpu
from jax.experimental.pallas import tpu_sc as plsc

N_PROMPT = 10
B = 1024
L = 210
LT = L - N_PROMPT
D = 128
NC, NS = 2, 16
NW = NC * NS
BPW = B // NW
HALF = LT // 2
PROWS = BPW * N_PROMPT
PIDX_COLS = 80
PIDX_RPW = PROWS // PIDX_COLS


def _build():
    mesh = plsc.VectorSubcoreMesh(core_axis_name="c", subcore_axis_name="s")

    @functools.partial(
        pl.kernel,
        mesh=mesh,
        out_type=jax.ShapeDtypeStruct((B * L, D), jnp.float32),
        compiler_params=pltpu.CompilerParams(use_tc_tiling_on_sc=False),
        scratch_types=[
            pltpu.VMEM((2 * BPW, HALF), jnp.int32),
            pltpu.VMEM((PIDX_RPW, PIDX_COLS), jnp.int32),
            pltpu.VMEM((PROWS, D), jnp.float32),
            pltpu.VMEM((LT, D), jnp.float32),
            pltpu.VMEM((LT, D), jnp.float32),
            pltpu.SemaphoreType.DMA,
            pltpu.SemaphoreType.DMA,
            pltpu.SemaphoreType.DMA,
            pltpu.SemaphoreType.DMA,
            pltpu.SemaphoreType.DMA,
        ],
    )
    def emb(tok_hbm, pidx_hbm, wte_hbm, pt_hbm, out_hbm,
            idx_v, pidx_v, pbuf, buf0, buf1,
            sem_p, sem_g0, sem_g1, sem_w0, sem_w1):
        wid = lax.axis_index("s") * NC + lax.axis_index("c")
        bufs = (buf0, buf1)
        sems_g = (sem_g0, sem_g1)
        sems_w = (sem_w0, sem_w1)

        def fire_gather(i, j):
            r0 = 2 * i
            pltpu.async_copy(wte_hbm.at[idx_v.at[r0]],
                             bufs[j].at[pl.ds(0, HALF)], sems_g[j])
            pltpu.async_copy(wte_hbm.at[idx_v.at[r0 + 1]],
                             bufs[j].at[pl.ds(HALF, HALF)], sems_g[j])

        def wait_gather(j):
            pltpu.make_async_copy(wte_hbm.at[idx_v.at[0]],
                                  bufs[j].at[pl.ds(0, HALF)],
                                  sems_g[j]).wait()
            pltpu.make_async_copy(wte_hbm.at[idx_v.at[1]],
                                  bufs[j].at[pl.ds(HALF, HALF)],
                                  sems_g[j]).wait()

        def fire_write(i, j):
            row = (wid * BPW + i) * L
            pltpu.async_copy(pbuf.at[pl.ds(i * N_PROMPT, N_PROMPT)],
                             out_hbm.at[pl.ds(row, N_PROMPT)], sems_w[j])
            pltpu.async_copy(bufs[j],
                             out_hbm.at[pl.ds(row + N_PROMPT, LT)], sems_w[j])

        def wait_write(j):
            pltpu.make_async_copy(pbuf.at[pl.ds(0, N_PROMPT)],
                                  out_hbm.at[pl.ds(0, N_PROMPT)],
                                  sems_w[j]).wait()
            pltpu.make_async_copy(bufs[j], out_hbm.at[pl.ds(0, LT)],
                                  sems_w[j]).wait()

        pltpu.sync_copy(tok_hbm.at[pl.ds(wid * 2 * BPW, 2 * BPW)], idx_v)
        pltpu.sync_copy(pidx_hbm.at[pl.ds(wid * PIDX_RPW, PIDX_RPW)], pidx_v)

        pcopies = [
            pltpu.async_copy(pt_hbm.at[pidx_v.at[r]],
                             pbuf.at[pl.ds(r * PIDX_COLS, PIDX_COLS)], sem_p)
            for r in range(PIDX_RPW)
        ]
        fire_gather(0, 0)
        for c in pcopies:
            c.wait()

        def step(i, j):
            @pl.when(i >= 1)
            def _():
                wait_write(1 - j)

            @pl.when(i + 1 < BPW)
            def _():
                fire_gather(i + 1, 1 - j)

            wait_gather(j)
            fire_write(i, j)

        def body(g, carry):
            step(2 * g, 0)
            step(2 * g + 1, 1)
            return carry

        lax.fori_loop(0, BPW // 2, body, 0)
        wait_write((BPW - 1) % 2)

    return emb


_EMB = _build()


def kernel(tokens, MoE_type_tensor, wte_weight,
           learned_embedding_text, learned_embedding_table):
    tok = tokens[:, N_PROMPT:].astype(jnp.int32).reshape(2 * B, HALF)
    m = MoE_type_tensor.astype(jnp.int32) * N_PROMPT
    pidx = (m[:, None] + jnp.arange(N_PROMPT, dtype=jnp.int32))
    pidx = pidx.reshape(B * N_PROMPT // PIDX_COLS, PIDX_COLS)
    ptable = jnp.concatenate([learned_embedding_text.astype(jnp.float32),
                              learned_embedding_table.astype(jnp.float32)],
                             axis=0)
    out = _EMB(tok, pidx, wte_weight.astype(jnp.float32), ptable)
    return out.reshape(B, L, D)


# speedup vs baseline: 4.2015x; 1.5856x over previous
import functools

import jax
import jax.numpy as jnp
from jax import lax
from jax.experimental import pallas as pl
from jax.experimental.pallas import tpu as pltpu
from jax.experimental.pallas import tpu_sc as plsc

N_PROMPT = 10
B = 1024
L = 210
LT = L - N_PROMPT
D = 128
NC, NS = 2, 16
NW = NC * NS
BPW = B // NW
HALF = LT // 2


def _build():
    mesh = plsc.VectorSubcoreMesh(core_axis_name="c", subcore_axis_name="s")

    @functools.partial(
        pl.kernel,
        mesh=mesh,
        out_type=jax.ShapeDtypeStruct((B, L, D), jnp.float32),
        scratch_types=[
            pltpu.VMEM((2 * BPW, HALF), jnp.int32),
            pltpu.VMEM((BPW, N_PROMPT), jnp.int32),
            pltpu.VMEM((L, D), jnp.float32),
            pltpu.VMEM((L, D), jnp.float32),
            pltpu.SemaphoreType.DMA,
            pltpu.SemaphoreType.DMA,
            pltpu.SemaphoreType.DMA,
            pltpu.SemaphoreType.DMA,
        ],
    )
    def emb(tok_hbm, pidx_hbm, wte_hbm, pt_hbm, out_hbm,
            idx_v, pidx_v, buf0, buf1,
            sem_g0, sem_g1, sem_w0, sem_w1):
        wid = lax.axis_index("s") * NC + lax.axis_index("c")
        bufs = (buf0, buf1)
        sems_g = (sem_g0, sem_g1)
        sems_w = (sem_w0, sem_w1)

        def fire_gather(i, j):
            pltpu.async_copy(pt_hbm.at[pidx_v.at[i]],
                             bufs[j].at[pl.ds(0, N_PROMPT)], sems_g[j])
            pltpu.async_copy(wte_hbm.at[idx_v.at[2 * i]],
                             bufs[j].at[pl.ds(N_PROMPT, HALF)], sems_g[j])
            pltpu.async_copy(wte_hbm.at[idx_v.at[2 * i + 1]],
                             bufs[j].at[pl.ds(N_PROMPT + HALF, HALF)],
                             sems_g[j])

        def wait_gather(j):
            pltpu.make_async_copy(pt_hbm.at[pidx_v.at[0]],
                                  bufs[j].at[pl.ds(0, N_PROMPT)],
                                  sems_g[j]).wait()
            pltpu.make_async_copy(wte_hbm.at[idx_v.at[0]],
                                  bufs[j].at[pl.ds(N_PROMPT, HALF)],
                                  sems_g[j]).wait()
            pltpu.make_async_copy(wte_hbm.at[idx_v.at[1]],
                                  bufs[j].at[pl.ds(N_PROMPT + HALF, HALF)],
                                  sems_g[j]).wait()

        def fire_write(i, j):
            pltpu.async_copy(bufs[j], out_hbm.at[wid * BPW + i], sems_w[j])

        def wait_write(j):
            pltpu.make_async_copy(bufs[j], out_hbm.at[0], sems_w[j]).wait()

        pltpu.sync_copy(tok_hbm.at[wid], idx_v)
        pltpu.sync_copy(pidx_hbm.at[wid], pidx_v)

        fire_gather(0, 0)

        def step(i, j):
            @pl.when(i >= 1)
            def _():
                wait_write(1 - j)

            @pl.when(i + 1 < BPW)
            def _():
                fire_gather(i + 1, 1 - j)

            wait_gather(j)
            fire_write(i, j)

        def body(g, carry):
            step(2 * g, 0)
            step(2 * g + 1, 1)
            return carry

        lax.fori_loop(0, BPW // 2, body, 0)
        wait_write((BPW - 1) % 2)

    return emb


_EMB = _build()


def kernel(tokens, MoE_type_tensor, wte_weight,
           learned_embedding_text, learned_embedding_table):
    tok = tokens[:, N_PROMPT:].astype(jnp.int32).reshape(NW, 2 * BPW, HALF)
    m = MoE_type_tensor.astype(jnp.int32) * N_PROMPT
    pidx = (m[:, None] + jnp.arange(N_PROMPT, dtype=jnp.int32))
    pidx = pidx.reshape(NW, BPW, N_PROMPT)
    ptable = jnp.concatenate([learned_embedding_text.astype(jnp.float32),
                              learned_embedding_table.astype(jnp.float32)],
                             axis=0)
    return _EMB(tok, pidx, wte_weight.astype(jnp.float32), ptable)
